# baseline (speedup 1.0000x reference)
import functools

import jax
import jax.numpy as jnp
from jax import lax
from jax.experimental import pallas as pl
from jax.experimental.pallas import tpu as pltpu
from jax.experimental.pallas import tpu_sc as plsc

N = 819200
NOFF = 16384
TOTAL = 2 * N
NC = 1
NS = 16
NW = NC * NS
OCHUNK = NOFF // NW
LANES = 16

COLS = 512
ROWS = N // COLS
GRID = 2
BROWS = ROWS // GRID

_MESH = plsc.VectorSubcoreMesh(core_axis_name="c", subcore_axis_name="s",
                               num_cores=NC)


@functools.partial(
    pl.kernel,
    out_type=jax.ShapeDtypeStruct((2 * NOFF + 1,), jnp.int32),
    mesh=_MESH,
    scratch_types=[
        pltpu.VMEM((OCHUNK,), jnp.int32),
        pltpu.VMEM((OCHUNK + LANES,), jnp.int32),
        pltpu.SemaphoreType.DMA,
        pltpu.SemaphoreType.DMA,
        pltpu.SemaphoreType.DMA,
    ],
)
def _combine_offsets(off0, off1, out_off, bo0, bo1, ssem, g0, g1):
    wid = lax.axis_index("s") * NC + lax.axis_index("c")
    obase = wid * OCHUNK

    cg0 = pltpu.async_copy(off0.at[pl.ds(obase, OCHUNK)], bo0, g0)
    cg1 = pltpu.async_copy(off1.at[pl.ds(obase, OCHUNK)],
                           bo1.at[pl.ds(0, OCHUNK)], g1)

    cg0.wait()
    so = pltpu.async_copy(bo0, out_off.at[pl.ds(obase, OCHUNK)], ssem)

    cg1.wait()

    def _rebase(i, carry):
        osl = pl.ds(i * LANES, LANES)
        bo1[osl] = bo1[osl] + jnp.int32(N)
        return carry

    lax.fori_loop(0, OCHUNK // LANES, _rebase, 0)
    bo1[pl.ds(OCHUNK, LANES)] = jnp.full((LANES,), TOTAL, dtype=jnp.int32)

    s1 = pltpu.async_copy(bo1.at[pl.ds(0, OCHUNK)],
                          out_off.at[pl.ds(NOFF + obase, OCHUNK)], ssem)

    @pl.when(wid == NW - 1)
    def _():
        pltpu.async_copy(bo1.at[pl.ds(OCHUNK, 1)],
                         out_off.at[pl.ds(2 * NOFF, 1)], ssem).wait()

    so.wait()
    s1.wait()


def _concat_body(i0_ref, i1_ref, w0_ref, w1_ref, ci_ref, cw_ref):
    ci_ref[0] = i0_ref[...]
    ci_ref[1] = i1_ref[...]
    cw_ref[0] = w0_ref[...]
    cw_ref[1] = w1_ref[...]


_concat_streams = pl.pallas_call(
    _concat_body,
    grid=(GRID,),
    in_specs=[
        pl.BlockSpec((BROWS, COLS), lambda i: (i, 0)),
        pl.BlockSpec((BROWS, COLS), lambda i: (i, 0)),
        pl.BlockSpec((BROWS, COLS), lambda i: (i, 0)),
        pl.BlockSpec((BROWS, COLS), lambda i: (i, 0)),
    ],
    out_specs=[
        pl.BlockSpec((2, BROWS, COLS), lambda i: (0, i, 0)),
        pl.BlockSpec((2, BROWS, COLS), lambda i: (0, i, 0)),
    ],
    out_shape=[
        jax.ShapeDtypeStruct((2, ROWS, COLS), jnp.int32),
        jax.ShapeDtypeStruct((2, ROWS, COLS), jnp.float32),
    ],
)


def kernel(indices_0, indices_1, offsets_0, offsets_1,
           per_sample_weights_0, per_sample_weights_1):
    combined_offsets = _combine_offsets(offsets_0, offsets_1)
    ci, cw = _concat_streams(
        indices_0.astype(jnp.int32).reshape(ROWS, COLS),
        indices_1.astype(jnp.int32).reshape(ROWS, COLS),
        per_sample_weights_0.reshape(ROWS, COLS),
        per_sample_weights_1.reshape(ROWS, COLS),
    )
    return ci.reshape(TOTAL), combined_offsets, cw.reshape(TOTAL)


# speedup vs baseline: 2.6019x; 2.6019x over previous
import functools

import jax
import jax.numpy as jnp
from jax import lax
from jax.experimental import pallas as pl
from jax.experimental.pallas import tpu as pltpu
from jax.experimental.pallas import tpu_sc as plsc

N = 819200
NOFF = 16384
TOTAL = 2 * N
NC = 1
NS = 16
NW = NC * NS
OCHUNK = NOFF // NW
LANES = 16

COLS = 128
ROWS = N // COLS
GRID = 2
BROWS = ROWS // GRID

_MESH = plsc.VectorSubcoreMesh(core_axis_name="c", subcore_axis_name="s",
                               num_cores=NC)


@functools.partial(
    pl.kernel,
    out_type=jax.ShapeDtypeStruct((2 * NOFF + 1,), jnp.int32),
    mesh=_MESH,
    scratch_types=[
        pltpu.VMEM((OCHUNK,), jnp.int32),
        pltpu.VMEM((OCHUNK + LANES,), jnp.int32),
        pltpu.SemaphoreType.DMA,
        pltpu.SemaphoreType.DMA,
        pltpu.SemaphoreType.DMA,
    ],
)
def _combine_offsets(off0, off1, out_off, bo0, bo1, ssem, g0, g1):
    wid = lax.axis_index("s") * NC + lax.axis_index("c")
    obase = wid * OCHUNK

    cg0 = pltpu.async_copy(off0.at[pl.ds(obase, OCHUNK)], bo0, g0)
    cg1 = pltpu.async_copy(off1.at[pl.ds(obase, OCHUNK)],
                           bo1.at[pl.ds(0, OCHUNK)], g1)

    cg0.wait()
    so = pltpu.async_copy(bo0, out_off.at[pl.ds(obase, OCHUNK)], ssem)

    cg1.wait()

    def _rebase(i, carry):
        osl = pl.ds(i * LANES, LANES)
        bo1[osl] = bo1[osl] + jnp.int32(N)
        return carry

    lax.fori_loop(0, OCHUNK // LANES, _rebase, 0)
    bo1[pl.ds(OCHUNK, LANES)] = jnp.full((LANES,), TOTAL, dtype=jnp.int32)

    s1 = pltpu.async_copy(bo1.at[pl.ds(0, OCHUNK)],
                          out_off.at[pl.ds(NOFF + obase, OCHUNK)], ssem)

    @pl.when(wid == NW - 1)
    def _():
        pltpu.async_copy(bo1.at[pl.ds(OCHUNK, 1)],
                         out_off.at[pl.ds(2 * NOFF, 1)], ssem).wait()

    so.wait()
    s1.wait()


def _concat_body(i0_ref, i1_ref, w0_ref, w1_ref, ci_ref, cw_ref):
    ci_ref[0] = i0_ref[...]
    ci_ref[1] = i1_ref[...]
    cw_ref[0] = w0_ref[...]
    cw_ref[1] = w1_ref[...]


_concat_streams = pl.pallas_call(
    _concat_body,
    grid=(GRID,),
    in_specs=[
        pl.BlockSpec((BROWS, COLS), lambda i: (i, 0)),
        pl.BlockSpec((BROWS, COLS), lambda i: (i, 0)),
        pl.BlockSpec((BROWS, COLS), lambda i: (i, 0)),
        pl.BlockSpec((BROWS, COLS), lambda i: (i, 0)),
    ],
    out_specs=[
        pl.BlockSpec((2, BROWS, COLS), lambda i: (0, i, 0)),
        pl.BlockSpec((2, BROWS, COLS), lambda i: (0, i, 0)),
    ],
    out_shape=[
        jax.ShapeDtypeStruct((2, ROWS, COLS), jnp.int32),
        jax.ShapeDtypeStruct((2, ROWS, COLS), jnp.float32),
    ],
)


def kernel(indices_0, indices_1, offsets_0, offsets_1,
           per_sample_weights_0, per_sample_weights_1):
    combined_offsets = _combine_offsets(offsets_0, offsets_1)
    ci, cw = _concat_streams(
        indices_0.astype(jnp.int32).reshape(ROWS, COLS),
        indices_1.astype(jnp.int32).reshape(ROWS, COLS),
        per_sample_weights_0.reshape(ROWS, COLS),
        per_sample_weights_1.reshape(ROWS, COLS),
    )
    return ci.reshape(TOTAL), combined_offsets, cw.reshape(TOTAL)
